# Initial kernel scaffold; baseline (speedup 1.0000x reference)
import jax
import jax.numpy as jnp
from jax.experimental import pallas as pl


def kernel(x):
    raise NotImplementedError("write your pallas kernel here")


# speedup vs baseline: 3.1088x; 3.1088x over previous
import functools

import numpy as np
import jax
import jax.numpy as jnp
from jax.experimental import pallas as pl
from jax.experimental.pallas import tpu as pltpu

_BS = 64
_NB = 4096 // _BS


def _tables():
    n = 16
    p = (np.arange(n) + 0.5) / n
    q = jax.scipy.special.ndtri(jnp.asarray(p, dtype=jnp.float32))
    q = q / jnp.max(jnp.abs(q))
    t16 = np.asarray(q.astype(jnp.float16))
    t32 = t16.astype(np.float32)
    u = (t32 + np.float32(1.0)) / np.float32(2.0)
    mids = (t32[:-1] + t32[1:]) * np.float32(0.5)
    c = (mids + np.float32(1.0)) * np.float32(0.5)
    return t32, u, c


_T32, _U, _C = _tables()


def _body(x_ref, o_ref, *, rows):
    x = x_ref[...]
    xb = x.reshape(rows, _NB, _BS)
    xmin = jnp.min(xb, axis=2)
    xmax = jnp.max(xb, axis=2)
    scales = xmax - xmin
    d = scales + jnp.float32(1e-8)
    smin = jnp.min(scales, axis=1, keepdims=True)
    smax = jnp.max(scales, axis=1, keepdims=True)
    ds = smax - smin + jnp.float32(1e-8)
    qs = jnp.round((scales - smin) / ds * jnp.float32(255.0))
    scales_d = smin + qs / jnp.float32(255.0) * (smax - smin)

    y = xb - xmin[:, :, None]
    u = jnp.full(xb.shape, _U[0], dtype=jnp.float32)
    for i in range(15):
        thr = d * jnp.float32(_C[i])
        u = jnp.where(y > thr[:, :, None], jnp.float32(_U[i + 1]), u)
    w = u * scales_d[:, :, None] + xmin[:, :, None]
    o_ref[...] = w.reshape(rows, _NB * _BS)


@jax.jit
def kernel(x):
    B, C = x.shape
    rows = 8
    grid = (B // rows,)
    return pl.pallas_call(
        functools.partial(_body, rows=rows),
        grid=grid,
        in_specs=[pl.BlockSpec((rows, C), lambda i: (i, 0))],
        out_specs=pl.BlockSpec((rows, C), lambda i: (i, 0)),
        out_shape=jax.ShapeDtypeStruct((B, C), jnp.float32),
    )(x)
